# baseline (speedup 1.0000x reference)
import functools

import jax
import jax.numpy as jnp
from jax import lax
from jax.experimental import pallas as pl
from jax.experimental.pallas import tpu as pltpu
from jax.experimental.pallas import tpu_sc as plsc

N = 49152
H = 64
NUM_MOD = 3
CHUNK = N // NUM_MOD

NC = 2
NS = 16
NW = NC * NS
PER_W = N // NW
CH = 128
NCH = PER_W // CH

NP = N // 2
PRB = 2048
_NB = (CHUNK // 2) // PRB


_NBT = NP // PRB


def _mm_body(xta_ref, xtb_ref, wa_ref, wb_ref, o_ref):
    ya = jax.lax.dot_general(
        xta_ref[...], wa_ref[0], (((0,), (0,)), ((), ())),
        preferred_element_type=jnp.float32)
    yb = jax.lax.dot_general(
        xtb_ref[...], wb_ref[0], (((0,), (0,)), ((), ())),
        preferred_element_type=jnp.float32)
    o_ref[:, 0:H] = ya
    o_ref[:, H:2 * H] = yb


def _expert_matmul(xt, wt):
    return pl.pallas_call(
        _mm_body,
        grid=(_NBT,),
        in_specs=[
            pl.BlockSpec((H, PRB), lambda b: (0, b)),
            pl.BlockSpec((H, PRB), lambda b: (0, b + _NBT)),
            pl.BlockSpec((1, H, H), lambda b: (b * PRB // CHUNK, 0, 0)),
            pl.BlockSpec((1, H, H), lambda b: ((NP + b * PRB) // CHUNK, 0, 0)),
        ],
        out_specs=pl.BlockSpec((PRB, 2 * H), lambda b: (b, 0)),
        out_shape=jax.ShapeDtypeStruct((NP, 2 * H), jnp.float32),
    )(xt, xt, wt, wt)


_sc_mesh = plsc.VectorSubcoreMesh(core_axis_name="c", subcore_axis_name="s")


HW = PER_W // 2


@functools.partial(
    pl.kernel,
    mesh=_sc_mesh,
    compiler_params=pltpu.CompilerParams(use_tc_tiling_on_sc=False),
    out_type=jax.ShapeDtypeStruct((N, H), jnp.float32),
    scratch_types=[
        pltpu.VMEM((PER_W,), jnp.int32),
        pltpu.VMEM((PER_W,), jnp.int32),
        pltpu.VMEM((PER_W,), jnp.int32),
        pltpu.VMEM((PER_W, H), jnp.float32),
        pltpu.SemaphoreType.DMA,
        pltpu.SemaphoreType.DMA,
    ],
)
def _sc_permute(perm_hbm, inv_hbm, y_hbm, out_hbm, perm_v, idx_v, idxp_v,
                rows_v, sem_idx, sem_rows):
    wid = lax.axis_index("s") * NC + lax.axis_index("c")
    pltpu.sync_copy(perm_hbm.at[pl.ds(wid * PER_W, PER_W)], perm_v)
    idx_copies = [
        pltpu.async_copy(inv_hbm.at[perm_v.at[pl.ds(j * CH, CH)]],
                         idx_v.at[pl.ds(j * CH, CH)], sem_idx)
        for j in range(NCH)
    ]
    for c in idx_copies:
        c.wait()

    def _remap(k, carry):
        v = idx_v[pl.ds(k * 16, 16)]
        idxp_v[pl.ds(k * 16, 16)] = jnp.where(v >= NP, 2 * v - (2 * NP - 1),
                                              2 * v)
        return carry

    lax.fori_loop(0, PER_W // 16, _remap, 0)
    row_copies = [
        pltpu.async_copy(y_hbm.at[idxp_v.at[pl.ds(j * CH, CH)]],
                         rows_v.at[pl.ds(j * CH, CH)], sem_rows)
        for j in range(NCH)
    ]
    for c in row_copies:
        c.wait()
    pltpu.sync_copy(rows_v, out_hbm.at[pl.ds(wid * PER_W, PER_W)])


def _tr_body(og_ref, o_ref):
    blk = og_ref[...]
    o_ref[:, 0:HW] = jnp.transpose(blk[:, 0:H])
    o_ref[:, HW:PER_W] = jnp.transpose(blk[:, H:2 * H])


def _untranspose(og):
    return pl.pallas_call(
        _tr_body,
        grid=(NW,),
        in_specs=[pl.BlockSpec((HW, 2 * H), lambda b: (b, 0))],
        out_specs=pl.BlockSpec((H, PER_W), lambda b: (0, b)),
        out_shape=jax.ShapeDtypeStruct((H, N), jnp.float32),
    )(og)


def kernel(x, permute_mapping, inv_permute_mapping, W0, W1, W2):
    wt = jnp.stack([W0.T, W1.T, W2.T])
    xt = x.T
    yh = _expert_matmul(xt, wt)
    y = yh.reshape(N, H)
    perm_r = permute_mapping.reshape(NW, 2, HW).swapaxes(1, 2).reshape(N)
    og64 = _sc_permute(perm_r, inv_permute_mapping, y)
    og = og64.reshape(NP, 2 * H)
    out_t = _untranspose(og)
    return out_t.T


# speedup vs baseline: 1.2063x; 1.2063x over previous
import functools

import jax
import jax.numpy as jnp
from jax import lax
from jax.experimental import pallas as pl
from jax.experimental.pallas import tpu as pltpu
from jax.experimental.pallas import tpu_sc as plsc

N = 49152
H = 64
NUM_MOD = 3
CHUNK = N // NUM_MOD

NC = 2
NS = 16
NW = NC * NS
PER_W = N // NW
CH = 128
NCH = PER_W // CH

NP = N // 2
PRB = 2048
_NB = (CHUNK // 2) // PRB


_NBT = NP // PRB


def _mm_body(xta_ref, xtb_ref, wa_ref, wb_ref, o_ref):
    ya = jax.lax.dot_general(
        xta_ref[...], wa_ref[0], (((0,), (0,)), ((), ())),
        preferred_element_type=jnp.float32)
    yb = jax.lax.dot_general(
        xtb_ref[...], wb_ref[0], (((0,), (0,)), ((), ())),
        preferred_element_type=jnp.float32)
    o_ref[:, 0:H] = ya
    o_ref[:, H:2 * H] = yb


def _expert_matmul(xt, wt):
    return pl.pallas_call(
        _mm_body,
        grid=(_NBT,),
        in_specs=[
            pl.BlockSpec((H, PRB), lambda b: (0, b)),
            pl.BlockSpec((H, PRB), lambda b: (0, b + _NBT)),
            pl.BlockSpec((1, H, H), lambda b: (b * PRB // CHUNK, 0, 0)),
            pl.BlockSpec((1, H, H), lambda b: ((NP + b * PRB) // CHUNK, 0, 0)),
        ],
        out_specs=pl.BlockSpec((PRB, 2 * H), lambda b: (b, 0)),
        out_shape=jax.ShapeDtypeStruct((NP, 2 * H), jnp.float32),
    )(xt, xt, wt, wt)


_sc_mesh = plsc.VectorSubcoreMesh(core_axis_name="c", subcore_axis_name="s")


HW = PER_W // 2


@functools.partial(
    pl.kernel,
    mesh=_sc_mesh,
    compiler_params=pltpu.CompilerParams(use_tc_tiling_on_sc=False),
    out_type=jax.ShapeDtypeStruct((N, H), jnp.float32),
    scratch_types=[
        pltpu.VMEM((PER_W,), jnp.int32),
        pltpu.VMEM((PER_W,), jnp.int32),
        pltpu.VMEM((PER_W,), jnp.int32),
        pltpu.VMEM((PER_W,), jnp.int32),
        pltpu.VMEM((PER_W, H), jnp.float32),
        pltpu.SemaphoreType.DMA,
        pltpu.SemaphoreType.DMA,
    ],
)
def _sc_permute(perm_hbm, inv_hbm, y_hbm, out_hbm, riff_v, perm_v, idx_v,
                idxp_v, rows_v, sem_idx, sem_rows):
    wid = lax.axis_index("s") * NC + lax.axis_index("c")
    base = wid * PER_W

    def _riff(k, carry):
        r = k * 16 + lax.iota(jnp.int32, 16)
        riff_v[pl.ds(k * 16, 16)] = base + HW * (r & 1) + (r >> 1)
        return carry

    lax.fori_loop(0, PER_W // 16, _riff, 0)
    perm_copies = [
        pltpu.async_copy(perm_hbm.at[riff_v.at[pl.ds(j * CH, CH)]],
                         perm_v.at[pl.ds(j * CH, CH)], sem_idx)
        for j in range(NCH)
    ]
    for c in perm_copies:
        c.wait()
    idx_copies = [
        pltpu.async_copy(inv_hbm.at[perm_v.at[pl.ds(j * CH, CH)]],
                         idx_v.at[pl.ds(j * CH, CH)], sem_idx)
        for j in range(NCH)
    ]
    for c in idx_copies:
        c.wait()

    def _remap(k, carry):
        v = idx_v[pl.ds(k * 16, 16)]
        idxp_v[pl.ds(k * 16, 16)] = jnp.where(v >= NP, 2 * v - (2 * NP - 1),
                                              2 * v)
        return carry

    lax.fori_loop(0, PER_W // 16, _remap, 0)
    row_copies = [
        pltpu.async_copy(y_hbm.at[idxp_v.at[pl.ds(j * CH, CH)]],
                         rows_v.at[pl.ds(j * CH, CH)], sem_rows)
        for j in range(NCH)
    ]
    for c in row_copies:
        c.wait()
    pltpu.sync_copy(rows_v, out_hbm.at[pl.ds(wid * PER_W, PER_W)])


def _tr_body(og_ref, eye_ref, o_ref):
    blk = og_ref[...]
    eye = eye_ref[...]
    o_ref[:, 0:HW] = jax.lax.dot_general(
        eye, blk[:, 0:H], (((1,), (1,)), ((), ())),
        preferred_element_type=jnp.float32)
    o_ref[:, HW:PER_W] = jax.lax.dot_general(
        eye, blk[:, H:2 * H], (((1,), (1,)), ((), ())),
        preferred_element_type=jnp.float32)


def _untranspose(og):
    eye = jnp.eye(H, dtype=jnp.float32)
    return pl.pallas_call(
        _tr_body,
        grid=(NW,),
        in_specs=[
            pl.BlockSpec((HW, 2 * H), lambda b: (b, 0)),
            pl.BlockSpec((H, H), lambda b: (0, 0)),
        ],
        out_specs=pl.BlockSpec((H, PER_W), lambda b: (0, b)),
        out_shape=jax.ShapeDtypeStruct((H, N), jnp.float32),
    )(og, eye)


def kernel(x, permute_mapping, inv_permute_mapping, W0, W1, W2):
    wt = jnp.stack([W0.T, W1.T, W2.T])
    xt = x.T
    yh = _expert_matmul(xt, wt)
    y = yh.reshape(N, H)
    og64 = _sc_permute(permute_mapping, inv_permute_mapping, y)
    og = og64.reshape(NP, 2 * H)
    out_t = _untranspose(og)
    return out_t.T


# speedup vs baseline: 1.5064x; 1.2488x over previous
import functools

import jax
import jax.numpy as jnp
from jax import lax
from jax.experimental import pallas as pl
from jax.experimental.pallas import tpu as pltpu
from jax.experimental.pallas import tpu_sc as plsc

N = 49152
H = 64
NUM_MOD = 3
CHUNK = N // NUM_MOD

NC = 2
NS = 16
NW = NC * NS
PER_W = N // NW
CH = 128
NCH = PER_W // CH

NP = N // 2
PRB = 4096
_NB = (CHUNK // 2) // PRB


_NBT = NP // PRB


def _mm_body(xta_ref, xtb_ref, wa_ref, wb_ref, o_ref):
    ya = jax.lax.dot_general(
        xta_ref[...], wa_ref[0], (((0,), (0,)), ((), ())),
        preferred_element_type=jnp.float32)
    yb = jax.lax.dot_general(
        xtb_ref[...], wb_ref[0], (((0,), (0,)), ((), ())),
        preferred_element_type=jnp.float32)
    o_ref[:, 0:H] = ya
    o_ref[:, H:2 * H] = yb


def _expert_matmul(xt, wt):
    return pl.pallas_call(
        _mm_body,
        grid=(_NBT,),
        in_specs=[
            pl.BlockSpec((H, PRB), lambda b: (0, b)),
            pl.BlockSpec((H, PRB), lambda b: (0, b + _NBT)),
            pl.BlockSpec((1, H, H), lambda b: (b * PRB // CHUNK, 0, 0)),
            pl.BlockSpec((1, H, H), lambda b: ((NP + b * PRB) // CHUNK, 0, 0)),
        ],
        out_specs=pl.BlockSpec((PRB, 2 * H), lambda b: (b, 0)),
        out_shape=jax.ShapeDtypeStruct((NP, 2 * H), jnp.float32),
    )(xt, xt, wt, wt)


_sc_mesh = plsc.VectorSubcoreMesh(core_axis_name="c", subcore_axis_name="s")


HW = PER_W // 2


@functools.partial(
    pl.kernel,
    mesh=_sc_mesh,
    compiler_params=pltpu.CompilerParams(use_tc_tiling_on_sc=False),
    out_type=jax.ShapeDtypeStruct((N, H), jnp.float32),
    scratch_types=[
        pltpu.VMEM((PER_W,), jnp.int32),
        pltpu.VMEM((PER_W,), jnp.int32),
        pltpu.VMEM((PER_W,), jnp.int32),
        pltpu.VMEM((PER_W,), jnp.int32),
        pltpu.VMEM((PER_W, H), jnp.float32),
        pltpu.SemaphoreType.DMA,
        pltpu.SemaphoreType.DMA,
    ],
)
def _sc_permute(perm_hbm, inv_hbm, y_hbm, out_hbm, riff_v, perm_v, idx_v,
                idxp_v, rows_v, sem_idx, sem_rows):
    wid = lax.axis_index("s") * NC + lax.axis_index("c")
    base = wid * PER_W

    def _riff(k, carry):
        r = k * 16 + lax.iota(jnp.int32, 16)
        riff_v[pl.ds(k * 16, 16)] = base + HW * (r & 1) + (r >> 1)
        return carry

    lax.fori_loop(0, PER_W // 16, _riff, 0)
    perm_copies = [
        pltpu.async_copy(perm_hbm.at[riff_v.at[pl.ds(j * CH, CH)]],
                         perm_v.at[pl.ds(j * CH, CH)], sem_idx)
        for j in range(NCH)
    ]
    for c in perm_copies:
        c.wait()
    idx_copies = [
        pltpu.async_copy(inv_hbm.at[perm_v.at[pl.ds(j * CH, CH)]],
                         idx_v.at[pl.ds(j * CH, CH)], sem_idx)
        for j in range(NCH)
    ]
    for c in idx_copies:
        c.wait()

    def _remap(k, carry):
        v = idx_v[pl.ds(k * 16, 16)]
        idxp_v[pl.ds(k * 16, 16)] = jnp.where(v >= NP, 2 * v - (2 * NP - 1),
                                              2 * v)
        return carry

    lax.fori_loop(0, PER_W // 16, _remap, 0)
    row_copies = [
        pltpu.async_copy(y_hbm.at[idxp_v.at[pl.ds(j * CH, CH)]],
                         rows_v.at[pl.ds(j * CH, CH)], sem_rows)
        for j in range(NCH)
    ]
    for c in row_copies:
        c.wait()
    pltpu.sync_copy(rows_v, out_hbm.at[pl.ds(wid * PER_W, PER_W)])


TB = 4


def _tr_body(og_ref, eye_ref, o_ref):
    eye = eye_ref[...]
    for s in range(TB):
        blk = og_ref[pl.ds(s * HW, HW), :]
        o_ref[:, s * PER_W:s * PER_W + HW] = jax.lax.dot_general(
            eye, blk[:, 0:H], (((1,), (1,)), ((), ())),
            preferred_element_type=jnp.float32)
        o_ref[:, s * PER_W + HW:(s + 1) * PER_W] = jax.lax.dot_general(
            eye, blk[:, H:2 * H], (((1,), (1,)), ((), ())),
            preferred_element_type=jnp.float32)


def _untranspose(og):
    eye = jnp.eye(H, dtype=jnp.float32)
    return pl.pallas_call(
        _tr_body,
        grid=(NW // TB,),
        in_specs=[
            pl.BlockSpec((TB * HW, 2 * H), lambda b: (b, 0)),
            pl.BlockSpec((H, H), lambda b: (0, 0)),
        ],
        out_specs=pl.BlockSpec((H, TB * PER_W), lambda b: (0, b)),
        out_shape=jax.ShapeDtypeStruct((H, N), jnp.float32),
    )(og, eye)


def kernel(x, permute_mapping, inv_permute_mapping, W0, W1, W2):
    wt = jnp.stack([W0.T, W1.T, W2.T])
    xt = x.T
    yh = _expert_matmul(xt, wt)
    y = yh.reshape(N, H)
    og64 = _sc_permute(permute_mapping, inv_permute_mapping, y)
    og = og64.reshape(NP, 2 * H)
    out_t = _untranspose(og)
    return out_t.T


# speedup vs baseline: 1.5891x; 1.0549x over previous
import functools

import jax
import jax.numpy as jnp
from jax import lax
from jax.experimental import pallas as pl
from jax.experimental.pallas import tpu as pltpu
from jax.experimental.pallas import tpu_sc as plsc

N = 49152
H = 64
NUM_MOD = 3
CHUNK = N // NUM_MOD

NC = 2
NS = 16
NW = NC * NS
PER_W = N // NW
CH = 128
NCH = PER_W // CH

NP = N // 2
PRB = 8192


_NBT = NP // PRB


def _pick_w(e, w0, w1, w2):
    return jnp.where(e == 0, w0, jnp.where(e == 1, w1, w2))


def _mm_body(xta_ref, xtb_ref, w0_ref, w1_ref, w2_ref, o_ref):
    b = pl.program_id(0)
    wa = _pick_w(b * PRB // CHUNK, w0_ref[...], w1_ref[...], w2_ref[...])
    wb = _pick_w((NP + b * PRB) // CHUNK, w0_ref[...], w1_ref[...],
                 w2_ref[...])
    ya = jax.lax.dot_general(
        xta_ref[...], wa, (((0,), (1,)), ((), ())),
        preferred_element_type=jnp.float32)
    yb = jax.lax.dot_general(
        xtb_ref[...], wb, (((0,), (1,)), ((), ())),
        preferred_element_type=jnp.float32)
    o_ref[:, 0:H] = ya
    o_ref[:, H:2 * H] = yb


def _expert_matmul(xt, w0, w1, w2):
    wspec = pl.BlockSpec((H, H), lambda b: (0, 0))
    return pl.pallas_call(
        _mm_body,
        grid=(_NBT,),
        in_specs=[
            pl.BlockSpec((H, PRB), lambda b: (0, b)),
            pl.BlockSpec((H, PRB), lambda b: (0, b + _NBT)),
            wspec, wspec, wspec,
        ],
        out_specs=pl.BlockSpec((PRB, 2 * H), lambda b: (b, 0)),
        out_shape=jax.ShapeDtypeStruct((NP, 2 * H), jnp.float32),
    )(xt, xt, w0, w1, w2)


_sc_mesh = plsc.VectorSubcoreMesh(core_axis_name="c", subcore_axis_name="s")


HW = PER_W // 2


@functools.partial(
    pl.kernel,
    mesh=_sc_mesh,
    compiler_params=pltpu.CompilerParams(use_tc_tiling_on_sc=False),
    out_type=jax.ShapeDtypeStruct((N, H), jnp.float32),
    scratch_types=[
        pltpu.VMEM((PER_W,), jnp.int32),
        pltpu.VMEM((PER_W,), jnp.int32),
        pltpu.VMEM((PER_W,), jnp.int32),
        pltpu.VMEM((PER_W,), jnp.int32),
        pltpu.VMEM((PER_W, H), jnp.float32),
        pltpu.SemaphoreType.DMA,
        pltpu.SemaphoreType.DMA,
    ],
)
def _sc_permute(perm_hbm, inv_hbm, y_hbm, out_hbm, riff_v, perm_v, idx_v,
                idxp_v, rows_v, sem_idx, sem_rows):
    wid = lax.axis_index("s") * NC + lax.axis_index("c")
    base = wid * PER_W

    def _riff(k, carry):
        r = k * 16 + lax.iota(jnp.int32, 16)
        riff_v[pl.ds(k * 16, 16)] = base + HW * (r & 1) + (r >> 1)
        return carry

    lax.fori_loop(0, PER_W // 16, _riff, 0)
    perm_copies = [
        pltpu.async_copy(perm_hbm.at[riff_v.at[pl.ds(j * CH, CH)]],
                         perm_v.at[pl.ds(j * CH, CH)], sem_idx)
        for j in range(NCH)
    ]
    for c in perm_copies:
        c.wait()
    idx_copies = [
        pltpu.async_copy(inv_hbm.at[perm_v.at[pl.ds(j * CH, CH)]],
                         idx_v.at[pl.ds(j * CH, CH)], sem_idx)
        for j in range(NCH)
    ]
    for c in idx_copies:
        c.wait()

    def _remap(k, carry):
        v = idx_v[pl.ds(k * 16, 16)]
        idxp_v[pl.ds(k * 16, 16)] = jnp.where(v >= NP, 2 * v - (2 * NP - 1),
                                              2 * v)
        return carry

    lax.fori_loop(0, PER_W // 16, _remap, 0)
    row_copies = [
        pltpu.async_copy(y_hbm.at[idxp_v.at[pl.ds(j * CH, CH)]],
                         rows_v.at[pl.ds(j * CH, CH)], sem_rows)
        for j in range(NCH)
    ]
    for c in row_copies:
        c.wait()
    pltpu.sync_copy(rows_v, out_hbm.at[pl.ds(wid * PER_W, PER_W)])


TB = 4


def _tr_body(og_ref, eye_ref, o_ref):
    eye = eye_ref[...]
    for s in range(TB):
        blk = og_ref[pl.ds(s * HW, HW), :]
        o_ref[:, s * PER_W:s * PER_W + HW] = jax.lax.dot_general(
            eye, blk[:, 0:H], (((1,), (1,)), ((), ())),
            preferred_element_type=jnp.float32)
        o_ref[:, s * PER_W + HW:(s + 1) * PER_W] = jax.lax.dot_general(
            eye, blk[:, H:2 * H], (((1,), (1,)), ((), ())),
            preferred_element_type=jnp.float32)


def _untranspose(og):
    eye = jnp.eye(H, dtype=jnp.float32)
    return pl.pallas_call(
        _tr_body,
        grid=(NW // TB,),
        in_specs=[
            pl.BlockSpec((TB * HW, 2 * H), lambda b: (b, 0)),
            pl.BlockSpec((H, H), lambda b: (0, 0)),
        ],
        out_specs=pl.BlockSpec((H, TB * PER_W), lambda b: (0, b)),
        out_shape=jax.ShapeDtypeStruct((H, N), jnp.float32),
    )(og, eye)


def kernel(x, permute_mapping, inv_permute_mapping, W0, W1, W2):
    xt = x.T
    yh = _expert_matmul(xt, W0, W1, W2)
    y = yh.reshape(N, H)
    og64 = _sc_permute(permute_mapping, inv_permute_mapping, y)
    og = og64.reshape(NP, 2 * H)
    out_t = _untranspose(og)
    return out_t.T


# speedup vs baseline: 1.6461x; 1.0359x over previous
import functools

import jax
import jax.numpy as jnp
from jax import lax
from jax.experimental import pallas as pl
from jax.experimental.pallas import tpu as pltpu
from jax.experimental.pallas import tpu_sc as plsc

N = 49152
H = 64
NUM_MOD = 3
CHUNK = N // NUM_MOD

NC = 2
NS = 16
NW = NC * NS
PER_W = N // NW
CH = 128
NCH = PER_W // CH

NP = N // 2
PRB = 8192


_NBT = NP // PRB


def _pick_w(e, w0, w1, w2):
    return jnp.where(e == 0, w0, jnp.where(e == 1, w1, w2))


def _mm_body(xta_ref, xtb_ref, w0_ref, w1_ref, w2_ref, o_ref):
    b = pl.program_id(0)
    wa = _pick_w(b * PRB // CHUNK, w0_ref[...], w1_ref[...], w2_ref[...])
    wb = _pick_w((NP + b * PRB) // CHUNK, w0_ref[...], w1_ref[...],
                 w2_ref[...])
    ya = jax.lax.dot_general(
        xta_ref[...], wa, (((0,), (1,)), ((), ())),
        preferred_element_type=jnp.float32)
    yb = jax.lax.dot_general(
        xtb_ref[...], wb, (((0,), (1,)), ((), ())),
        preferred_element_type=jnp.float32)
    o_ref[:, 0:H] = ya
    o_ref[:, H:2 * H] = yb


def _expert_matmul(xt, w0, w1, w2):
    wspec = pl.BlockSpec((H, H), lambda b: (0, 0))
    return pl.pallas_call(
        _mm_body,
        grid=(_NBT,),
        in_specs=[
            pl.BlockSpec((H, PRB), lambda b: (0, b)),
            pl.BlockSpec((H, PRB), lambda b: (0, b + _NBT)),
            wspec, wspec, wspec,
        ],
        out_specs=pl.BlockSpec((PRB, 2 * H), lambda b: (b, 0)),
        out_shape=jax.ShapeDtypeStruct((NP, 2 * H), jnp.float32),
    )(xt, xt, w0, w1, w2)


_sc_mesh = plsc.VectorSubcoreMesh(core_axis_name="c", subcore_axis_name="s")


HW = PER_W // 2


@functools.partial(
    pl.kernel,
    mesh=_sc_mesh,
    compiler_params=pltpu.CompilerParams(use_tc_tiling_on_sc=False),
    out_type=jax.ShapeDtypeStruct((N,), jnp.int32),
    scratch_types=[
        pltpu.VMEM((PER_W,), jnp.int32),
        pltpu.VMEM((PER_W,), jnp.int32),
        pltpu.VMEM((PER_W,), jnp.int32),
        pltpu.VMEM((PER_W,), jnp.int32),
        pltpu.SemaphoreType.DMA,
    ],
)
def _sc_compose(perm_hbm, inv_hbm, out_hbm, riff_v, perm_v, idx_v, idxp_v,
                sem_idx):
    wid = lax.axis_index("s") * NC + lax.axis_index("c")
    base = wid * PER_W

    def _riff(k, carry):
        r = k * 16 + lax.iota(jnp.int32, 16)
        riff_v[pl.ds(k * 16, 16)] = base + HW * (r & 1) + (r >> 1)
        return carry

    lax.fori_loop(0, PER_W // 16, _riff, 0)
    perm_copies = [
        pltpu.async_copy(perm_hbm.at[riff_v.at[pl.ds(j * CH, CH)]],
                         perm_v.at[pl.ds(j * CH, CH)], sem_idx)
        for j in range(NCH)
    ]
    for c in perm_copies:
        c.wait()
    idx_copies = [
        pltpu.async_copy(inv_hbm.at[perm_v.at[pl.ds(j * CH, CH)]],
                         idx_v.at[pl.ds(j * CH, CH)], sem_idx)
        for j in range(NCH)
    ]
    for c in idx_copies:
        c.wait()

    def _remap(k, carry):
        v = idx_v[pl.ds(k * 16, 16)]
        idxp_v[pl.ds(k * 16, 16)] = jnp.where(v >= NP, 2 * v - (2 * NP - 1),
                                              2 * v)
        return carry

    lax.fori_loop(0, PER_W // 16, _remap, 0)
    pltpu.sync_copy(idxp_v, out_hbm.at[pl.ds(base, PER_W)])


@functools.partial(
    pl.kernel,
    mesh=_sc_mesh,
    compiler_params=pltpu.CompilerParams(use_tc_tiling_on_sc=False),
    out_type=jax.ShapeDtypeStruct((N, H), jnp.float32),
    scratch_types=[
        pltpu.VMEM((PER_W,), jnp.int32),
        pltpu.VMEM((PER_W, H), jnp.float32),
        pltpu.SemaphoreType.DMA,
    ],
)
def _sc_gather(idxp_hbm, y_hbm, out_hbm, idxp_v, rows_v, sem_rows):
    wid = lax.axis_index("s") * NC + lax.axis_index("c")
    pltpu.sync_copy(idxp_hbm.at[pl.ds(wid * PER_W, PER_W)], idxp_v)
    row_copies = [
        pltpu.async_copy(y_hbm.at[idxp_v.at[pl.ds(j * CH, CH)]],
                         rows_v.at[pl.ds(j * CH, CH)], sem_rows)
        for j in range(NCH)
    ]
    for c in row_copies:
        c.wait()
    pltpu.sync_copy(rows_v, out_hbm.at[pl.ds(wid * PER_W, PER_W)])


TB = 4


def _tr_body(og_ref, eye_ref, o_ref):
    eye = eye_ref[...]
    for s in range(TB):
        blk = og_ref[pl.ds(s * HW, HW), :]
        o_ref[:, s * PER_W:s * PER_W + HW] = jax.lax.dot_general(
            eye, blk[:, 0:H], (((1,), (1,)), ((), ())),
            preferred_element_type=jnp.float32)
        o_ref[:, s * PER_W + HW:(s + 1) * PER_W] = jax.lax.dot_general(
            eye, blk[:, H:2 * H], (((1,), (1,)), ((), ())),
            preferred_element_type=jnp.float32)


def _untranspose(og):
    eye = jnp.eye(H, dtype=jnp.float32)
    return pl.pallas_call(
        _tr_body,
        grid=(NW // TB,),
        in_specs=[
            pl.BlockSpec((TB * HW, 2 * H), lambda b: (b, 0)),
            pl.BlockSpec((H, H), lambda b: (0, 0)),
        ],
        out_specs=pl.BlockSpec((H, TB * PER_W), lambda b: (0, b)),
        out_shape=jax.ShapeDtypeStruct((H, N), jnp.float32),
    )(og, eye)


def kernel(x, permute_mapping, inv_permute_mapping, W0, W1, W2):
    xt = x.T
    idxp = _sc_compose(permute_mapping, inv_permute_mapping)
    yh = _expert_matmul(xt, W0, W1, W2)
    y = yh.reshape(N, H)
    og64 = _sc_gather(idxp, y)
    og = og64.reshape(NP, 2 * H)
    out_t = _untranspose(og)
    return out_t.T


# speedup vs baseline: 1.6991x; 1.0322x over previous
import functools

import jax
import jax.numpy as jnp
from jax import lax
from jax.experimental import pallas as pl
from jax.experimental.pallas import tpu as pltpu
from jax.experimental.pallas import tpu_sc as plsc

N = 49152
H = 64
NUM_MOD = 3
CHUNK = N // NUM_MOD

NC = 2
NS = 16
NW = NC * NS
PER_W = N // NW
CH = 128
NCH = PER_W // CH

NP = N // 2
PRB = 8192


_NBT = NP // PRB


def _pick_w(e, w0, w1, w2):
    return jnp.where(e == 0, w0, jnp.where(e == 1, w1, w2))


def _mm_body(xta_ref, xtb_ref, w0_ref, w1_ref, w2_ref, o_ref):
    b = pl.program_id(0)
    wa = _pick_w(b * PRB // CHUNK, w0_ref[...], w1_ref[...], w2_ref[...])
    wb = _pick_w((NP + b * PRB) // CHUNK, w0_ref[...], w1_ref[...],
                 w2_ref[...])
    ya = jax.lax.dot_general(
        xta_ref[...], wa, (((0,), (1,)), ((), ())),
        preferred_element_type=jnp.float32)
    yb = jax.lax.dot_general(
        xtb_ref[...], wb, (((0,), (1,)), ((), ())),
        preferred_element_type=jnp.float32)
    o_ref[:, 0:H] = ya
    o_ref[:, H:2 * H] = yb


def _expert_matmul(xt, w0, w1, w2):
    wspec = pl.BlockSpec((H, H), lambda b: (0, 0))
    return pl.pallas_call(
        _mm_body,
        grid=(_NBT,),
        in_specs=[
            pl.BlockSpec((H, PRB), lambda b: (0, b)),
            pl.BlockSpec((H, PRB), lambda b: (0, b + _NBT)),
            wspec, wspec, wspec,
        ],
        out_specs=pl.BlockSpec((PRB, 2 * H), lambda b: (b, 0)),
        out_shape=jax.ShapeDtypeStruct((NP, 2 * H), jnp.float32),
    )(xt, xt, w0, w1, w2)


_sc_mesh = plsc.VectorSubcoreMesh(core_axis_name="c", subcore_axis_name="s")


HW = PER_W // 2


@functools.partial(
    pl.kernel,
    mesh=_sc_mesh,
    compiler_params=pltpu.CompilerParams(use_tc_tiling_on_sc=False),
    out_type=jax.ShapeDtypeStruct((N,), jnp.int32),
    scratch_types=[
        pltpu.VMEM((PER_W,), jnp.int32),
        pltpu.VMEM((PER_W,), jnp.int32),
        pltpu.VMEM((PER_W,), jnp.int32),
        pltpu.VMEM((PER_W,), jnp.int32),
        pltpu.SemaphoreType.DMA,
    ],
)
def _sc_compose(perm_hbm, inv_hbm, out_hbm, riff_v, perm_v, idx_v, idxp_v,
                sem_idx):
    wid = lax.axis_index("s") * NC + lax.axis_index("c")
    base = wid * PER_W

    def _riff(k, carry):
        r = k * 16 + lax.iota(jnp.int32, 16)
        riff_v[pl.ds(k * 16, 16)] = base + HW * (r & 1) + (r >> 1)
        return carry

    lax.fori_loop(0, PER_W // 16, _riff, 0)
    perm_copies = [
        pltpu.async_copy(perm_hbm.at[riff_v.at[pl.ds(j * CH, CH)]],
                         perm_v.at[pl.ds(j * CH, CH)], sem_idx)
        for j in range(NCH)
    ]
    for c in perm_copies:
        c.wait()
    idx_copies = [
        pltpu.async_copy(inv_hbm.at[perm_v.at[pl.ds(j * CH, CH)]],
                         idx_v.at[pl.ds(j * CH, CH)], sem_idx)
        for j in range(NCH)
    ]
    for c in idx_copies:
        c.wait()

    def _remap(k, carry):
        v = idx_v[pl.ds(k * 16, 16)]
        idxp_v[pl.ds(k * 16, 16)] = jnp.where(v >= NP, 2 * v - (2 * NP - 1),
                                              2 * v)
        return carry

    lax.fori_loop(0, PER_W // 16, _remap, 0)
    pltpu.sync_copy(idxp_v, out_hbm.at[pl.ds(base, PER_W)])


@functools.partial(
    pl.kernel,
    mesh=_sc_mesh,
    compiler_params=pltpu.CompilerParams(use_tc_tiling_on_sc=False),
    out_type=jax.ShapeDtypeStruct((N, H), jnp.float32),
    scratch_types=[
        pltpu.VMEM((PER_W,), jnp.int32),
        pltpu.VMEM((PER_W, H), jnp.float32),
        pltpu.SemaphoreType.DMA,
    ],
)
def _sc_gather(idxp_hbm, y_hbm, out_hbm, idxp_v, rows_v, sem_rows):
    wid = lax.axis_index("s") * NC + lax.axis_index("c")
    pltpu.sync_copy(idxp_hbm.at[pl.ds(wid * PER_W, PER_W)], idxp_v)
    row_copies = [
        pltpu.async_copy(y_hbm.at[idxp_v.at[pl.ds(j * CH, CH)]],
                         rows_v.at[pl.ds(j * CH, CH)], sem_rows)
        for j in range(NCH)
    ]
    for c in row_copies:
        c.wait()
    pltpu.sync_copy(rows_v, out_hbm.at[pl.ds(wid * PER_W, PER_W)])


TB = 8


def _tr_body(og_ref, eye_ref, o_ref):
    eye = eye_ref[...]
    for s in range(TB):
        blk = og_ref[pl.ds(s * HW, HW), :]
        o_ref[:, s * PER_W:s * PER_W + HW] = jax.lax.dot_general(
            eye, blk[:, 0:H], (((1,), (1,)), ((), ())),
            preferred_element_type=jnp.float32)
        o_ref[:, s * PER_W + HW:(s + 1) * PER_W] = jax.lax.dot_general(
            eye, blk[:, H:2 * H], (((1,), (1,)), ((), ())),
            preferred_element_type=jnp.float32)


def _untranspose(og):
    eye = jnp.eye(H, dtype=jnp.float32)
    return pl.pallas_call(
        _tr_body,
        grid=(NW // TB,),
        in_specs=[
            pl.BlockSpec((TB * HW, 2 * H), lambda b: (b, 0)),
            pl.BlockSpec((H, H), lambda b: (0, 0)),
        ],
        out_specs=pl.BlockSpec((H, TB * PER_W), lambda b: (0, b)),
        out_shape=jax.ShapeDtypeStruct((H, N), jnp.float32),
    )(og, eye)


def kernel(x, permute_mapping, inv_permute_mapping, W0, W1, W2):
    xt = x.T
    idxp = _sc_compose(permute_mapping, inv_permute_mapping)
    yh = _expert_matmul(xt, W0, W1, W2)
    y = yh.reshape(N, H)
    og64 = _sc_gather(idxp, y)
    og = og64.reshape(NP, 2 * H)
    out_t = _untranspose(og)
    return out_t.T


# speedup vs baseline: 1.7543x; 1.0325x over previous
import functools

import jax
import jax.numpy as jnp
from jax import lax
from jax.experimental import pallas as pl
from jax.experimental.pallas import tpu as pltpu
from jax.experimental.pallas import tpu_sc as plsc

N = 49152
H = 64
NUM_MOD = 3
CHUNK = N // NUM_MOD

NC = 2
NS = 16
NW = NC * NS
PER_W = N // NW
CH = 128
NCH = PER_W // CH

NP = N // 2
PRB = 8192


_NBT = NP // PRB


def _pick_w(e, w0, w1, w2):
    return jnp.where(e == 0, w0, jnp.where(e == 1, w1, w2))


def _mm_body(xta_ref, xtb_ref, w0_ref, w1_ref, w2_ref, o_ref):
    b = pl.program_id(0)
    wa = _pick_w(b * PRB // CHUNK, w0_ref[...], w1_ref[...], w2_ref[...])
    wb = _pick_w((NP + b * PRB) // CHUNK, w0_ref[...], w1_ref[...],
                 w2_ref[...])
    ya = jax.lax.dot_general(
        xta_ref[...], wa, (((0,), (1,)), ((), ())),
        preferred_element_type=jnp.float32)
    yb = jax.lax.dot_general(
        xtb_ref[...], wb, (((0,), (1,)), ((), ())),
        preferred_element_type=jnp.float32)
    o_ref[:, 0:H] = ya
    o_ref[:, H:2 * H] = yb


def _expert_matmul(xt, w0, w1, w2):
    wspec = pl.BlockSpec((H, H), lambda b: (0, 0))
    return pl.pallas_call(
        _mm_body,
        grid=(_NBT,),
        in_specs=[
            pl.BlockSpec((H, PRB), lambda b: (0, b)),
            pl.BlockSpec((H, PRB), lambda b: (0, b + _NBT)),
            wspec, wspec, wspec,
        ],
        out_specs=pl.BlockSpec((PRB, 2 * H), lambda b: (b, 0)),
        out_shape=jax.ShapeDtypeStruct((NP, 2 * H), jnp.float32),
    )(xt, xt, w0, w1, w2)


_sc_mesh = plsc.VectorSubcoreMesh(core_axis_name="c", subcore_axis_name="s")


HW = PER_W // 2


@functools.partial(
    pl.kernel,
    mesh=_sc_mesh,
    compiler_params=pltpu.CompilerParams(use_tc_tiling_on_sc=False),
    out_type=jax.ShapeDtypeStruct((N,), jnp.int32),
    scratch_types=[
        pltpu.VMEM((PER_W,), jnp.int32),
        pltpu.VMEM((PER_W,), jnp.int32),
        pltpu.VMEM((PER_W,), jnp.int32),
        pltpu.VMEM((PER_W,), jnp.int32),
        pltpu.SemaphoreType.DMA,
    ],
)
def _sc_compose(perm_hbm, inv_hbm, out_hbm, riff_v, perm_v, idx_v, idxp_v,
                sem_idx):
    wid = lax.axis_index("s") * NC + lax.axis_index("c")
    base = wid * PER_W

    def _riff(k, carry):
        r = k * 16 + lax.iota(jnp.int32, 16)
        riff_v[pl.ds(k * 16, 16)] = base + HW * (r & 1) + (r >> 1)
        return carry

    lax.fori_loop(0, PER_W // 16, _riff, 0)
    perm_copies = [
        pltpu.async_copy(perm_hbm.at[riff_v.at[pl.ds(j * CH, CH)]],
                         perm_v.at[pl.ds(j * CH, CH)], sem_idx)
        for j in range(NCH)
    ]
    for c in perm_copies:
        c.wait()
    idx_copies = [
        pltpu.async_copy(inv_hbm.at[perm_v.at[pl.ds(j * CH, CH)]],
                         idx_v.at[pl.ds(j * CH, CH)], sem_idx)
        for j in range(NCH)
    ]
    for c in idx_copies:
        c.wait()

    def _remap(k, carry):
        v = idx_v[pl.ds(k * 16, 16)]
        idxp_v[pl.ds(k * 16, 16)] = jnp.where(v >= NP, 2 * v - (2 * NP - 1),
                                              2 * v)
        return carry

    lax.fori_loop(0, PER_W // 16, _remap, 0)
    pltpu.sync_copy(idxp_v, out_hbm.at[pl.ds(base, PER_W)])


@functools.partial(
    pl.kernel,
    mesh=_sc_mesh,
    compiler_params=pltpu.CompilerParams(use_tc_tiling_on_sc=False),
    out_type=jax.ShapeDtypeStruct((N, H), jnp.float32),
    scratch_types=[
        pltpu.VMEM((PER_W,), jnp.int32),
        pltpu.VMEM((PER_W, H), jnp.float32),
        pltpu.SemaphoreType.DMA,
    ],
)
def _sc_gather(idxp_hbm, y_hbm, out_hbm, idxp_v, rows_v, sem_rows):
    wid = lax.axis_index("s") * NC + lax.axis_index("c")
    pltpu.sync_copy(idxp_hbm.at[pl.ds(wid * PER_W, PER_W)], idxp_v)
    row_copies = [
        pltpu.async_copy(y_hbm.at[idxp_v.at[pl.ds(j * CH, CH)]],
                         rows_v.at[pl.ds(j * CH, CH)], sem_rows)
        for j in range(NCH)
    ]
    for c in row_copies:
        c.wait()
    pltpu.sync_copy(rows_v, out_hbm.at[pl.ds(wid * PER_W, PER_W)])


TB = 16


def _tr_body(og_ref, eye_ref, o_ref):
    eye = eye_ref[...]
    for s in range(TB):
        blk = og_ref[pl.ds(s * HW, HW), :]
        o_ref[:, s * PER_W:s * PER_W + HW] = jax.lax.dot_general(
            eye, blk[:, 0:H], (((1,), (1,)), ((), ())),
            preferred_element_type=jnp.float32)
        o_ref[:, s * PER_W + HW:(s + 1) * PER_W] = jax.lax.dot_general(
            eye, blk[:, H:2 * H], (((1,), (1,)), ((), ())),
            preferred_element_type=jnp.float32)


def _untranspose(og):
    eye = jnp.eye(H, dtype=jnp.float32)
    return pl.pallas_call(
        _tr_body,
        grid=(NW // TB,),
        in_specs=[
            pl.BlockSpec((TB * HW, 2 * H), lambda b: (b, 0)),
            pl.BlockSpec((H, H), lambda b: (0, 0)),
        ],
        out_specs=pl.BlockSpec((H, TB * PER_W), lambda b: (0, b)),
        out_shape=jax.ShapeDtypeStruct((H, N), jnp.float32),
    )(og, eye)


def kernel(x, permute_mapping, inv_permute_mapping, W0, W1, W2):
    xt = x.T
    idxp = _sc_compose(permute_mapping, inv_permute_mapping)
    yh = _expert_matmul(xt, W0, W1, W2)
    y = yh.reshape(N, H)
    og64 = _sc_gather(idxp, y)
    og = og64.reshape(NP, 2 * H)
    out_t = _untranspose(og)
    return out_t.T
